# Initial kernel scaffold; baseline (speedup 1.0000x reference)
import functools

import jax
import jax.numpy as jnp
from jax import lax
from jax.experimental import pallas as pl
from jax.experimental.pallas import tpu as pltpu
from jax.experimental.pallas import tpu_sc as plsc

_N = 10000
_E = 320000
_D = 128

_NC = 2
_NS = 16
_NW = _NC * _NS
_EPW = _E // _NW
_CH = 80
_NCHUNK = _EPW // _CH
_RPT = _N // _NS
_ZR = 125
_CW = 16


def _zero_vmem(ref, rows, cols):
    zv = jnp.zeros((16,), jnp.float32)

    def row(i, _):
        def col(j, __):
            ref[i, pl.ds(j * 16, 16)] = zv
            return 0
        return lax.fori_loop(0, cols // 16, col, 0)

    lax.fori_loop(0, rows, row, 0)


def _fill_ones(ref, rows, cols):
    ov = jnp.ones((16,), jnp.float32)

    def row(i, _):
        def col(j, __):
            ref[i, pl.ds(j * 16, 16)] = ov
            return 0
        return lax.fori_loop(0, cols // 16, col, 0)

    lax.fori_loop(0, rows, row, 0)


def _sc_agg_body(with_cnt, x_hbm, edges_hbm, *rest):
    if with_cnt:
        (out_hbm, cnt_hbm, src_v, dst_v, rows_v, ones_v, zbuf, zcnt,
         acc, cnt_acc, sem) = rest
    else:
        (out_hbm, src_v, dst_v, rows_v, zbuf, acc, sem) = rest

    c = lax.axis_index("c")
    s = lax.axis_index("s")
    wid = s * _NC + c

    _zero_vmem(zbuf, _ZR, _D)
    for k in range(_RPT // _ZR):
        pltpu.sync_copy(zbuf, acc.at[pl.ds(s * _RPT + k * _ZR, _ZR)])
    if with_cnt:
        _zero_vmem(zcnt, _ZR, _CW)
        _fill_ones(ones_v, _CH, _CW)
        for k in range(_RPT // _ZR):
            pltpu.sync_copy(zcnt, cnt_acc.at[pl.ds(s * _RPT + k * _ZR, _ZR)])
    plsc.subcore_barrier()

    def chunk(i, _):
        base = pl.multiple_of(wid * _EPW + i * _CH, 8)
        pltpu.sync_copy(edges_hbm.at[0, pl.ds(base, _CH)], src_v)
        pltpu.sync_copy(edges_hbm.at[1, pl.ds(base, _CH)], dst_v)
        pltpu.async_copy(x_hbm.at[src_v], rows_v, sem).wait()
        pltpu.sync_copy(rows_v, acc.at[dst_v], add=True)
        if with_cnt:
            pltpu.sync_copy(ones_v, cnt_acc.at[dst_v], add=True)
        return 0

    lax.fori_loop(0, _NCHUNK, chunk, 0)
    plsc.subcore_barrier()

    pltpu.sync_copy(acc.at[pl.ds(s * _RPT, _RPT)],
                    out_hbm.at[c, pl.ds(s * _RPT, _RPT)])
    if with_cnt:
        pltpu.sync_copy(cnt_acc.at[pl.ds(s * _RPT, _RPT)],
                        cnt_hbm.at[c, pl.ds(s * _RPT, _RPT)])


def _make_sc_agg(with_cnt):
    mesh = plsc.VectorSubcoreMesh(core_axis_name="c", subcore_axis_name="s")
    out_type = [jax.ShapeDtypeStruct((_NC, _N, _D), jnp.float32)]
    scratch = [
        pltpu.VMEM((_CH,), jnp.int32),
        pltpu.VMEM((_CH,), jnp.int32),
        pltpu.VMEM((_CH, _D), jnp.float32),
    ]
    if with_cnt:
        out_type.append(jax.ShapeDtypeStruct((_NC, _N, _CW), jnp.float32))
        scratch.append(pltpu.VMEM((_CH, _CW), jnp.float32))
    scratch.append(pltpu.VMEM((_ZR, _D), jnp.float32))
    if with_cnt:
        scratch.append(pltpu.VMEM((_ZR, _CW), jnp.float32))
    scratch.append(pltpu.VMEM_SHARED((_N, _D), jnp.float32))
    if with_cnt:
        scratch.append(pltpu.VMEM_SHARED((_N, _CW), jnp.float32))
    scratch.append(pltpu.SemaphoreType.DMA)

    return pl.kernel(
        functools.partial(_sc_agg_body, with_cnt),
        out_type=tuple(out_type) if with_cnt else out_type[0],
        mesh=mesh,
        scratch_types=scratch,
    )


_sc_agg_with_cnt = _make_sc_agg(True)
_sc_agg = _make_sc_agg(False)


_RB = 2000


def _tc_layer1_body(x_ref, p_ref, cnt_ref, wl_ref, bl_ref, wr_ref, h_ref):
    cnt = cnt_ref[0, :, 0:1] + cnt_ref[1, :, 0:1]
    mean = (p_ref[0] + p_ref[1]) / jnp.maximum(cnt, 1.0)
    dn = (((1,), (1,)), ((), ()))
    h = (lax.dot_general(x_ref[...], wl_ref[...], dn,
                         preferred_element_type=jnp.float32)
         + bl_ref[...]
         + lax.dot_general(mean, wr_ref[...], dn,
                           preferred_element_type=jnp.float32))
    h_ref[...] = jnp.maximum(h, 0.0)


def _tc_layer2_body(h_ref, p_ref, cnt_ref, wl_ref, bl_ref, wr_ref,
                    wo_ref, bo_ref, out_ref):
    cnt = cnt_ref[0, :, 0:1] + cnt_ref[1, :, 0:1]
    mean = (p_ref[0] + p_ref[1]) / jnp.maximum(cnt, 1.0)
    dn = (((1,), (1,)), ((), ()))
    h2 = (lax.dot_general(h_ref[...], wl_ref[...], dn,
                          preferred_element_type=jnp.float32)
          + bl_ref[...]
          + lax.dot_general(mean, wr_ref[...], dn,
                            preferred_element_type=jnp.float32))
    h2 = jnp.maximum(h2, 0.0)
    out_ref[...] = lax.dot_general(h2, wo_ref[...], dn,
                                   preferred_element_type=jnp.float32) + bo_ref[...]


def _row_spec():
    return pl.BlockSpec((_RB, _D), lambda i: (i, 0))


def _part_spec():
    return pl.BlockSpec((_NC, _RB, _D), lambda i: (0, i, 0))


def _cnt_spec():
    return pl.BlockSpec((_NC, _RB, _CW), lambda i: (0, i, 0))


def _w_spec():
    return pl.BlockSpec((_D, _D), lambda i: (0, 0))


def _b_spec():
    return pl.BlockSpec((_D,), lambda i: (0,))


def _tc_layer1(x, p, cntp, Wl, bl, Wr):
    return pl.pallas_call(
        _tc_layer1_body,
        grid=(_N // _RB,),
        in_specs=[_row_spec(), _part_spec(), _cnt_spec(),
                  _w_spec(), _b_spec(), _w_spec()],
        out_specs=_row_spec(),
        out_shape=jax.ShapeDtypeStruct((_N, _D), jnp.float32),
    )(x, p, cntp, Wl, bl, Wr)


def _tc_layer2(h, p, cntp, Wl, bl, Wr, Wo, bo):
    return pl.pallas_call(
        _tc_layer2_body,
        grid=(_N // _RB,),
        in_specs=[_row_spec(), _part_spec(), _cnt_spec(),
                  _w_spec(), _b_spec(), _w_spec(), _w_spec(), _b_spec()],
        out_specs=_row_spec(),
        out_shape=jax.ShapeDtypeStruct((_N, _D), jnp.float32),
    )(h, p, cntp, Wl, bl, Wr, Wo, bo)


def kernel(x, edge_index, W1l, b1l, W1r, W2l, b2l, W2r, Wout, bout):
    p1, cntp = _sc_agg_with_cnt(x, edge_index)
    h = _tc_layer1(x, p1, cntp, W1l, b1l, W1r)
    p2 = _sc_agg(h, edge_index)
    return _tc_layer2(h, p2, cntp, W2l, b2l, W2r, Wout, bout)


# speedup vs baseline: 4.9313x; 4.9313x over previous
import jax
import jax.numpy as jnp
from jax import lax
from jax.experimental import pallas as pl
from jax.experimental.pallas import tpu as pltpu
from jax.experimental.pallas import tpu_sc as plsc

_N = 10000
_E = 320000
_D = 128

_NC = 2
_NS = 16
_NW = _NC * _NS
_EPW = _E // _NW
_CH = 80
_NCHUNK = _EPW // _CH
_NP = 10240
_RPT = _NP // _NS
_ZR = 64


def _fill_vmem(ref, rows, cols, value):
    v = jnp.full((16,), value, jnp.float32)

    def row(i, _):
        def col(j, __):
            ref[i, pl.ds(j * 16, 16)] = v
            return 0
        return lax.fori_loop(0, cols // 16, col, 0)

    lax.fori_loop(0, rows, row, 0)


def _zero_acc(zbuf, acc, s):
    _fill_vmem(zbuf, _ZR, _D, 0.0)
    for k in range(_RPT // _ZR):
        pltpu.sync_copy(zbuf, acc.at[pl.ds(s * _RPT + k * _ZR, _ZR)])


def _write_out(acc, out_hbm, c, s):
    pltpu.sync_copy(acc.at[pl.ds(s * _RPT, _RPT)],
                    out_hbm.at[c, pl.ds(s * _RPT, _RPT)])


def _sc_agg_body(x_hbm, edges_hbm, out_hbm, src_v, dst_v, rows_v, zbuf,
                 acc, sem):
    c = lax.axis_index("c")
    s = lax.axis_index("s")
    wid = s * _NC + c

    _zero_acc(zbuf, acc, s)
    plsc.subcore_barrier()

    def chunk(i, _):
        base = pl.multiple_of(wid * _EPW + i * _CH, 8)
        pltpu.sync_copy(edges_hbm.at[pl.ds(base, _CH)], src_v)
        pltpu.sync_copy(edges_hbm.at[pl.ds(_E + base, _CH)], dst_v)
        pltpu.async_copy(x_hbm.at[src_v], rows_v, sem).wait()
        pltpu.sync_copy(rows_v, acc.at[dst_v], add=True)
        return 0

    lax.fori_loop(0, _NCHUNK, chunk, 0)
    plsc.subcore_barrier()

    _write_out(acc, out_hbm, c, s)


def _sc_cnt_body(edges_hbm, out_hbm, dst_v, ones_v, zbuf, acc):
    c = lax.axis_index("c")
    s = lax.axis_index("s")
    wid = s * _NC + c

    _zero_acc(zbuf, acc, s)
    _fill_vmem(ones_v, _CH, _D, 1.0)
    plsc.subcore_barrier()

    def chunk(i, _):
        base = pl.multiple_of(wid * _EPW + i * _CH, 8)
        pltpu.sync_copy(edges_hbm.at[pl.ds(_E + base, _CH)], dst_v)
        pltpu.sync_copy(ones_v, acc.at[dst_v], add=True)
        return 0

    lax.fori_loop(0, _NCHUNK, chunk, 0)
    plsc.subcore_barrier()

    _write_out(acc, out_hbm, c, s)


_sc_mesh = plsc.VectorSubcoreMesh(core_axis_name="c", subcore_axis_name="s")

_sc_agg = pl.kernel(
    _sc_agg_body,
    out_type=jax.ShapeDtypeStruct((_NC, _NP, _D), jnp.float32),
    mesh=_sc_mesh,
    scratch_types=[
        pltpu.VMEM((_CH,), jnp.int32),
        pltpu.VMEM((_CH,), jnp.int32),
        pltpu.VMEM((_CH, _D), jnp.float32),
        pltpu.VMEM((_ZR, _D), jnp.float32),
        pltpu.VMEM_SHARED((_NP, _D), jnp.float32),
        pltpu.SemaphoreType.DMA,
    ],
)

_sc_cnt = pl.kernel(
    _sc_cnt_body,
    out_type=jax.ShapeDtypeStruct((_NC, _NP, _D), jnp.float32),
    mesh=_sc_mesh,
    scratch_types=[
        pltpu.VMEM((_CH,), jnp.int32),
        pltpu.VMEM((_CH, _D), jnp.float32),
        pltpu.VMEM((_ZR, _D), jnp.float32),
        pltpu.VMEM_SHARED((_NP, _D), jnp.float32),
    ],
)


_RB = 2000


def _tc_layer1_body(x_ref, p_ref, cnt_ref, wl_ref, bl_ref, wr_ref, h_ref):
    cnt = cnt_ref[0, :, 0:1] + cnt_ref[1, :, 0:1]
    mean = (p_ref[0] + p_ref[1]) / jnp.maximum(cnt, 1.0)
    dn = (((1,), (1,)), ((), ()))
    h = (lax.dot_general(x_ref[...], wl_ref[...], dn,
                         preferred_element_type=jnp.float32)
         + bl_ref[...]
         + lax.dot_general(mean, wr_ref[...], dn,
                           preferred_element_type=jnp.float32))
    h_ref[...] = jnp.maximum(h, 0.0)


def _tc_layer2_body(h_ref, p_ref, cnt_ref, wl_ref, bl_ref, wr_ref,
                    wo_ref, bo_ref, out_ref):
    cnt = cnt_ref[0, :, 0:1] + cnt_ref[1, :, 0:1]
    mean = (p_ref[0] + p_ref[1]) / jnp.maximum(cnt, 1.0)
    dn = (((1,), (1,)), ((), ()))
    h2 = (lax.dot_general(h_ref[...], wl_ref[...], dn,
                          preferred_element_type=jnp.float32)
          + bl_ref[...]
          + lax.dot_general(mean, wr_ref[...], dn,
                            preferred_element_type=jnp.float32))
    h2 = jnp.maximum(h2, 0.0)
    out_ref[...] = lax.dot_general(h2, wo_ref[...], dn,
                                   preferred_element_type=jnp.float32) + bo_ref[...]


def _row_spec():
    return pl.BlockSpec((_RB, _D), lambda i: (i, 0))


def _part_spec():
    return pl.BlockSpec((_NC, _RB, _D), lambda i: (0, i, 0))


def _w_spec():
    return pl.BlockSpec((_D, _D), lambda i: (0, 0))


def _b_spec():
    return pl.BlockSpec((_D,), lambda i: (0,))


def _tc_layer1(x, p, cntp, Wl, bl, Wr):
    return pl.pallas_call(
        _tc_layer1_body,
        grid=(_N // _RB,),
        in_specs=[_row_spec(), _part_spec(), _part_spec(),
                  _w_spec(), _b_spec(), _w_spec()],
        out_specs=_row_spec(),
        out_shape=jax.ShapeDtypeStruct((_N, _D), jnp.float32),
    )(x, p, cntp, Wl, bl, Wr)


def _tc_layer2(h, p, cntp, Wl, bl, Wr, Wo, bo):
    return pl.pallas_call(
        _tc_layer2_body,
        grid=(_N // _RB,),
        in_specs=[_row_spec(), _part_spec(), _part_spec(),
                  _w_spec(), _b_spec(), _w_spec(), _w_spec(), _b_spec()],
        out_specs=_row_spec(),
        out_shape=jax.ShapeDtypeStruct((_N, _D), jnp.float32),
    )(h, p, cntp, Wl, bl, Wr, Wo, bo)


def kernel(x, edge_index, W1l, b1l, W1r, W2l, b2l, W2r, Wout, bout):
    edges_flat = edge_index.reshape(2 * _E)
    cntp = _sc_cnt(edges_flat)
    p1 = _sc_agg(x, edges_flat)
    h = _tc_layer1(x, p1, cntp, W1l, b1l, W1r)
    p2 = _sc_agg(h, edges_flat)
    return _tc_layer2(h, p2, cntp, W2l, b2l, W2r, Wout, bout)


# speedup vs baseline: 8.6396x; 1.7520x over previous
import jax
import jax.numpy as jnp
from jax import lax
from jax.experimental import pallas as pl
from jax.experimental.pallas import tpu as pltpu
from jax.experimental.pallas import tpu_sc as plsc

_N = 10000
_E = 320000
_D = 128

_NC = 2
_NS = 16
_NW = _NC * _NS
_CH = 80
_NCHUNK = 128
_EPWP = _CH * _NCHUNK
_NPAD = _EPWP - _E // _NW
_NP = 10240
_RPT = _NP // _NS
_ZR = 64


def _fill_vmem(ref, rows, cols, value):
    v = jnp.full((16,), value, jnp.float32)

    def row(i, _):
        def col(j, __):
            ref[i, pl.ds(j * 16, 16)] = v
            return 0
        return lax.fori_loop(0, cols // 16, col, 0)

    lax.fori_loop(0, rows, row, 0)


def _zero_acc(zbuf, acc, s):
    _fill_vmem(zbuf, _ZR, _D, 0.0)
    for k in range(_RPT // _ZR):
        pltpu.sync_copy(zbuf, acc.at[pl.ds(s * _RPT + k * _ZR, _ZR)])


def _write_out(acc, out_hbm, c, s):
    pltpu.sync_copy(acc.at[pl.ds(s * _RPT, _RPT)],
                    out_hbm.at[c, pl.ds(s * _RPT, _RPT)])


def _sc_agg_body(x_hbm, idx_hbm, out_hbm, ia, ib, ra, rb, zbuf, acc,
                 isem, gsem):
    c = lax.axis_index("c")
    s = lax.axis_index("s")
    wid = s * _NC + c

    _zero_acc(zbuf, acc, s)
    plsc.subcore_barrier()

    pltpu.sync_copy(idx_hbm.at[wid, 0], ia)
    pltpu.async_copy(x_hbm.at[ia.at[0]], ra, gsem)
    pltpu.async_copy(idx_hbm.at[wid, 1], ib, isem)

    def pair(i, _):
        k = i * 2
        pltpu.make_async_copy(idx_hbm.at[wid, 0], ib, isem).wait()
        pltpu.make_async_copy(x_hbm.at[ia.at[0]], ra, gsem).wait()
        pltpu.async_copy(x_hbm.at[ib.at[0]], rb, gsem)
        pltpu.async_copy(idx_hbm.at[wid, k + 2], ia, isem)
        pltpu.sync_copy(ra, acc.at[ia.at[1]], add=True)
        pltpu.make_async_copy(idx_hbm.at[wid, 0], ia, isem).wait()
        pltpu.make_async_copy(x_hbm.at[ib.at[0]], rb, gsem).wait()
        pltpu.async_copy(x_hbm.at[ia.at[0]], ra, gsem)
        pltpu.async_copy(idx_hbm.at[wid, k + 3], ib, isem)
        pltpu.sync_copy(rb, acc.at[ib.at[1]], add=True)
        return 0

    lax.fori_loop(0, _NCHUNK // 2, pair, 0)

    pltpu.make_async_copy(x_hbm.at[ia.at[0]], ra, gsem).wait()
    pltpu.make_async_copy(idx_hbm.at[wid, 0], ib, isem).wait()
    plsc.subcore_barrier()

    _write_out(acc, out_hbm, c, s)


def _sc_cnt_body(dst_hbm, out_hbm, dst_all, ones_v, zbuf, acc, ssem):
    c = lax.axis_index("c")
    s = lax.axis_index("s")
    wid = s * _NC + c

    _zero_acc(zbuf, acc, s)
    _fill_vmem(ones_v, _CH, _D, 1.0)
    pltpu.sync_copy(dst_hbm.at[wid], dst_all)
    plsc.subcore_barrier()

    pltpu.async_copy(ones_v, acc.at[dst_all.at[0]], ssem, add=True)

    def chunk(j, _):
        pltpu.async_copy(ones_v, acc.at[dst_all.at[j + 1]], ssem, add=True)
        pltpu.make_async_copy(ones_v, acc.at[dst_all.at[0]], ssem).wait()
        return 0

    lax.fori_loop(0, _NCHUNK - 1, chunk, 0)
    pltpu.make_async_copy(ones_v, acc.at[dst_all.at[0]], ssem).wait()
    plsc.subcore_barrier()

    _write_out(acc, out_hbm, c, s)


_sc_mesh = plsc.VectorSubcoreMesh(core_axis_name="c", subcore_axis_name="s")

_sc_agg = pl.kernel(
    _sc_agg_body,
    out_type=jax.ShapeDtypeStruct((_NC, _NP, _D), jnp.float32),
    mesh=_sc_mesh,
    scratch_types=[
        pltpu.VMEM((2, _CH), jnp.int32),
        pltpu.VMEM((2, _CH), jnp.int32),
        pltpu.VMEM((_CH, _D), jnp.float32),
        pltpu.VMEM((_CH, _D), jnp.float32),
        pltpu.VMEM((_ZR, _D), jnp.float32),
        pltpu.VMEM_SHARED((_NP, _D), jnp.float32),
        pltpu.SemaphoreType.DMA,
        pltpu.SemaphoreType.DMA,
    ],
)

_sc_cnt = pl.kernel(
    _sc_cnt_body,
    out_type=jax.ShapeDtypeStruct((_NC, _NP, _D), jnp.float32),
    mesh=_sc_mesh,
    scratch_types=[
        pltpu.VMEM((_NCHUNK, _CH), jnp.int32),
        pltpu.VMEM((_CH, _D), jnp.float32),
        pltpu.VMEM((_ZR, _D), jnp.float32),
        pltpu.VMEM_SHARED((_NP, _D), jnp.float32),
        pltpu.SemaphoreType.DMA,
    ],
)


def _pad_edges(edge_index):
    src = edge_index[0].reshape(_NW, _E // _NW)
    dst = edge_index[1].reshape(_NW, _E // _NW)
    pad_src = (jnp.arange(_NW * _NPAD, dtype=jnp.int32) % _N).reshape(_NW, _NPAD)
    pad_dst = (_N + jnp.arange(_NW * _NPAD, dtype=jnp.int32) % (_NP - _N)
               ).reshape(_NW, _NPAD)
    src = jnp.concatenate([src, pad_src], axis=1).reshape(_NW, _NCHUNK, _CH)
    dst = jnp.concatenate([dst, pad_dst], axis=1).reshape(_NW, _NCHUNK, _CH)
    idx4 = jnp.stack([src, dst], axis=2)
    idx4 = jnp.concatenate([idx4, idx4[:, :2]], axis=1)
    return idx4, dst


_RB = 2000


def _tc_layer1_body(x_ref, p_ref, cnt_ref, wl_ref, bl_ref, wr_ref, h_ref):
    cnt = cnt_ref[0, :, 0:1] + cnt_ref[1, :, 0:1]
    mean = (p_ref[0] + p_ref[1]) / jnp.maximum(cnt, 1.0)
    dn = (((1,), (1,)), ((), ()))
    h = (lax.dot_general(x_ref[...], wl_ref[...], dn,
                         preferred_element_type=jnp.float32)
         + bl_ref[...]
         + lax.dot_general(mean, wr_ref[...], dn,
                           preferred_element_type=jnp.float32))
    h_ref[...] = jnp.maximum(h, 0.0)


def _tc_layer2_body(h_ref, p_ref, cnt_ref, wl_ref, bl_ref, wr_ref,
                    wo_ref, bo_ref, out_ref):
    cnt = cnt_ref[0, :, 0:1] + cnt_ref[1, :, 0:1]
    mean = (p_ref[0] + p_ref[1]) / jnp.maximum(cnt, 1.0)
    dn = (((1,), (1,)), ((), ()))
    h2 = (lax.dot_general(h_ref[...], wl_ref[...], dn,
                          preferred_element_type=jnp.float32)
          + bl_ref[...]
          + lax.dot_general(mean, wr_ref[...], dn,
                            preferred_element_type=jnp.float32))
    h2 = jnp.maximum(h2, 0.0)
    out_ref[...] = lax.dot_general(h2, wo_ref[...], dn,
                                   preferred_element_type=jnp.float32) + bo_ref[...]


def _row_spec():
    return pl.BlockSpec((_RB, _D), lambda i: (i, 0))


def _part_spec():
    return pl.BlockSpec((_NC, _RB, _D), lambda i: (0, i, 0))


def _w_spec():
    return pl.BlockSpec((_D, _D), lambda i: (0, 0))


def _b_spec():
    return pl.BlockSpec((_D,), lambda i: (0,))


def _tc_layer1(x, p, cntp, Wl, bl, Wr):
    return pl.pallas_call(
        _tc_layer1_body,
        grid=(_N // _RB,),
        in_specs=[_row_spec(), _part_spec(), _part_spec(),
                  _w_spec(), _b_spec(), _w_spec()],
        out_specs=_row_spec(),
        out_shape=jax.ShapeDtypeStruct((_N, _D), jnp.float32),
    )(x, p, cntp, Wl, bl, Wr)


def _tc_layer2(h, p, cntp, Wl, bl, Wr, Wo, bo):
    return pl.pallas_call(
        _tc_layer2_body,
        grid=(_N // _RB,),
        in_specs=[_row_spec(), _part_spec(), _part_spec(),
                  _w_spec(), _b_spec(), _w_spec(), _w_spec(), _b_spec()],
        out_specs=_row_spec(),
        out_shape=jax.ShapeDtypeStruct((_N, _D), jnp.float32),
    )(h, p, cntp, Wl, bl, Wr, Wo, bo)


def kernel(x, edge_index, W1l, b1l, W1r, W2l, b2l, W2r, Wout, bout):
    idx4, dst3 = _pad_edges(edge_index)
    cntp = _sc_cnt(dst3)
    p1 = _sc_agg(x, idx4)
    h = _tc_layer1(x, p1, cntp, W1l, b1l, W1r)
    p2 = _sc_agg(h, idx4)
    return _tc_layer2(h, p2, cntp, W2l, b2l, W2r, Wout, bout)


# speedup vs baseline: 9.6252x; 1.1141x over previous
import functools

import jax
import jax.numpy as jnp
from jax import lax
from jax.experimental import pallas as pl
from jax.experimental.pallas import tpu as pltpu
from jax.experimental.pallas import tpu_sc as plsc

_N = 10000
_E = 320000
_D = 128

_NC = 2
_NS = 16
_NW = _NC * _NS
_CH = 128
_NCHUNK = 80
_EPWP = _CH * _NCHUNK
_NPAD = _EPWP - _E // _NW
_NP = 10240
_RPT = _NP // _NS


def _fill_vmem(ref, rows, cols, value):
    v = jnp.full((16,), value, jnp.float32)

    def row(i, _):
        def col(j, __):
            ref[i, pl.ds(j * 16, 16)] = v
            return 0
        return lax.fori_loop(0, cols // 16, col, 0)

    lax.fori_loop(0, rows, row, 0)


def _zero_acc(zbuf, acc, s):
    for k in range(_RPT // _CH):
        pltpu.sync_copy(zbuf, acc.at[pl.ds(s * _RPT + k * _CH, _CH)])


def _write_out(acc, out_hbm, c, s):
    pltpu.sync_copy(acc.at[pl.ds(s * _RPT, _RPT)],
                    out_hbm.at[c, pl.ds(s * _RPT, _RPT)])


def _cnt_phase(idx_hbm, cnt_hbm, ia, ib, ones_v, acc, isem, ssem, wid, c, s):
    pltpu.sync_copy(idx_hbm.at[wid, 0], ia)
    pltpu.sync_copy(idx_hbm.at[wid, 1], ib)

    def pair(i, _):
        k = i * 2
        pltpu.async_copy(ones_v, acc.at[ia.at[1]], ssem, add=True)
        pltpu.async_copy(ones_v, acc.at[ib.at[1]], ssem, add=True)
        pltpu.make_async_copy(ones_v, acc.at[ia.at[1]], ssem).wait()
        pltpu.async_copy(idx_hbm.at[wid, k + 2], ia, isem)
        pltpu.make_async_copy(ones_v, acc.at[ib.at[1]], ssem).wait()
        pltpu.async_copy(idx_hbm.at[wid, k + 3], ib, isem)
        pltpu.make_async_copy(idx_hbm.at[wid, 0], ia, isem).wait()
        pltpu.make_async_copy(idx_hbm.at[wid, 0], ib, isem).wait()
        return 0

    lax.fori_loop(0, _NCHUNK // 2, pair, 0)
    plsc.subcore_barrier()
    _write_out(acc, cnt_hbm, c, s)


def _agg_phase(x_hbm, idx_hbm, out_hbm, ia, ib, ra, rb, acc, isem, gsem,
               wid, c, s):
    pltpu.sync_copy(idx_hbm.at[wid, 0], ia)
    pltpu.async_copy(x_hbm.at[ia.at[0]], ra, gsem)
    pltpu.async_copy(idx_hbm.at[wid, 1], ib, isem)

    def pair(i, _):
        k = i * 2
        pltpu.make_async_copy(idx_hbm.at[wid, 0], ib, isem).wait()
        pltpu.make_async_copy(x_hbm.at[ia.at[0]], ra, gsem).wait()
        pltpu.async_copy(x_hbm.at[ib.at[0]], rb, gsem)
        pltpu.async_copy(idx_hbm.at[wid, k + 2], ia, isem)
        pltpu.sync_copy(ra, acc.at[ia.at[1]], add=True)
        pltpu.make_async_copy(idx_hbm.at[wid, 0], ia, isem).wait()
        pltpu.make_async_copy(x_hbm.at[ib.at[0]], rb, gsem).wait()
        pltpu.async_copy(x_hbm.at[ia.at[0]], ra, gsem)
        pltpu.async_copy(idx_hbm.at[wid, k + 3], ib, isem)
        pltpu.sync_copy(rb, acc.at[ib.at[1]], add=True)
        return 0

    lax.fori_loop(0, _NCHUNK // 2, pair, 0)

    pltpu.make_async_copy(x_hbm.at[ia.at[0]], ra, gsem).wait()
    pltpu.make_async_copy(idx_hbm.at[wid, 0], ib, isem).wait()
    plsc.subcore_barrier()
    _write_out(acc, out_hbm, c, s)


def _sc_agg_body(with_cnt, x_hbm, idx_hbm, *rest):
    if with_cnt:
        out_hbm, cnt_hbm, ia, ib, ra, rb, acc, isem, gsem = rest
    else:
        out_hbm, ia, ib, ra, rb, acc, isem, gsem = rest

    c = lax.axis_index("c")
    s = lax.axis_index("s")
    wid = s * _NC + c

    _fill_vmem(rb, _CH, _D, 0.0)
    _zero_acc(rb, acc, s)
    plsc.subcore_barrier()

    if with_cnt:
        _fill_vmem(ra, _CH, _D, 1.0)
        _cnt_phase(idx_hbm, cnt_hbm, ia, ib, ra, acc, isem, gsem, wid, c, s)
        plsc.subcore_barrier()
        _zero_acc(rb, acc, s)
        plsc.subcore_barrier()

    _agg_phase(x_hbm, idx_hbm, out_hbm, ia, ib, ra, rb, acc, isem, gsem,
               wid, c, s)


def _make_sc_agg(with_cnt):
    out_type = [jax.ShapeDtypeStruct((_NC, _NP, _D), jnp.float32)]
    if with_cnt:
        out_type.append(jax.ShapeDtypeStruct((_NC, _NP, _D), jnp.float32))
    return pl.kernel(
        functools.partial(_sc_agg_body, with_cnt),
        out_type=tuple(out_type) if with_cnt else out_type[0],
        mesh=plsc.VectorSubcoreMesh(core_axis_name="c", subcore_axis_name="s"),
        scratch_types=[
            pltpu.VMEM((2, _CH), jnp.int32),
            pltpu.VMEM((2, _CH), jnp.int32),
            pltpu.VMEM((_CH, _D), jnp.float32),
            pltpu.VMEM((_CH, _D), jnp.float32),
            pltpu.VMEM_SHARED((_NP, _D), jnp.float32),
            pltpu.SemaphoreType.DMA,
            pltpu.SemaphoreType.DMA,
        ],
    )


_sc_agg_with_cnt = _make_sc_agg(True)
_sc_agg = _make_sc_agg(False)


def _pad_edges(edge_index):
    src = edge_index[0].reshape(_NW, _E // _NW)
    dst = edge_index[1].reshape(_NW, _E // _NW)
    pad_src = (jnp.arange(_NW * _NPAD, dtype=jnp.int32) % _N).reshape(_NW, _NPAD)
    pad_dst = (_N + jnp.arange(_NW * _NPAD, dtype=jnp.int32) % (_NP - _N)
               ).reshape(_NW, _NPAD)
    src = jnp.concatenate([src, pad_src], axis=1).reshape(_NW, _NCHUNK, _CH)
    dst = jnp.concatenate([dst, pad_dst], axis=1).reshape(_NW, _NCHUNK, _CH)
    idx4 = jnp.stack([src, dst], axis=2)
    idx4 = jnp.concatenate([idx4, idx4[:, :2]], axis=1)
    return idx4


_RB = 2000


def _tc_layer1_body(x_ref, p_ref, cnt_ref, wl_ref, bl_ref, wr_ref, h_ref):
    cnt = cnt_ref[0, :, 0:1] + cnt_ref[1, :, 0:1]
    mean = (p_ref[0] + p_ref[1]) / jnp.maximum(cnt, 1.0)
    dn = (((1,), (1,)), ((), ()))
    h = (lax.dot_general(x_ref[...], wl_ref[...], dn,
                         preferred_element_type=jnp.float32)
         + bl_ref[...]
         + lax.dot_general(mean, wr_ref[...], dn,
                           preferred_element_type=jnp.float32))
    h_ref[...] = jnp.maximum(h, 0.0)


def _tc_layer2_body(h_ref, p_ref, cnt_ref, wl_ref, bl_ref, wr_ref,
                    wo_ref, bo_ref, out_ref):
    cnt = cnt_ref[0, :, 0:1] + cnt_ref[1, :, 0:1]
    mean = (p_ref[0] + p_ref[1]) / jnp.maximum(cnt, 1.0)
    dn = (((1,), (1,)), ((), ()))
    h2 = (lax.dot_general(h_ref[...], wl_ref[...], dn,
                          preferred_element_type=jnp.float32)
          + bl_ref[...]
          + lax.dot_general(mean, wr_ref[...], dn,
                            preferred_element_type=jnp.float32))
    h2 = jnp.maximum(h2, 0.0)
    out_ref[...] = lax.dot_general(h2, wo_ref[...], dn,
                                   preferred_element_type=jnp.float32) + bo_ref[...]


def _row_spec():
    return pl.BlockSpec((_RB, _D), lambda i: (i, 0))


def _part_spec():
    return pl.BlockSpec((_NC, _RB, _D), lambda i: (0, i, 0))


def _w_spec():
    return pl.BlockSpec((_D, _D), lambda i: (0, 0))


def _b_spec():
    return pl.BlockSpec((_D,), lambda i: (0,))


def _tc_layer1(x, p, cntp, Wl, bl, Wr):
    return pl.pallas_call(
        _tc_layer1_body,
        grid=(_N // _RB,),
        in_specs=[_row_spec(), _part_spec(), _part_spec(),
                  _w_spec(), _b_spec(), _w_spec()],
        out_specs=_row_spec(),
        out_shape=jax.ShapeDtypeStruct((_N, _D), jnp.float32),
    )(x, p, cntp, Wl, bl, Wr)


def _tc_layer2(h, p, cntp, Wl, bl, Wr, Wo, bo):
    return pl.pallas_call(
        _tc_layer2_body,
        grid=(_N // _RB,),
        in_specs=[_row_spec(), _part_spec(), _part_spec(),
                  _w_spec(), _b_spec(), _w_spec(), _w_spec(), _b_spec()],
        out_specs=_row_spec(),
        out_shape=jax.ShapeDtypeStruct((_N, _D), jnp.float32),
    )(h, p, cntp, Wl, bl, Wr, Wo, bo)


def kernel(x, edge_index, W1l, b1l, W1r, W2l, b2l, W2r, Wout, bout):
    idx4 = _pad_edges(edge_index)
    p1, cntp = _sc_agg_with_cnt(x, idx4)
    h = _tc_layer1(x, p1, cntp, W1l, b1l, W1r)
    p2 = _sc_agg(h, idx4)
    return _tc_layer2(h, p2, cntp, W2l, b2l, W2r, Wout, bout)
